# baseline (speedup 1.0000x reference)
import jax
import jax.numpy as jnp
from jax.experimental import pallas as pl
from jax.experimental.pallas import tpu as pltpu

_B, _L, _H = 2, 2048, 1024
_A, _R, _K = 8, 128, 2
_N = _B * _L
_BLK = 512
_W1ROWS = _A * _R + _A


def _fused_body(x_ref, w1a_ref, b1_ref, bg_ref, w2f_ref, b2_ref, out_ref):
    xb = x_ref[...].astype(jnp.bfloat16)
    hall = jax.lax.dot_general(
        xb, w1a_ref[...],
        dimension_numbers=(((1,), (1,)), ((), ())),
        preferred_element_type=jnp.float32,
    )
    logits = hall[:, _A * _R :] + bg_ref[...]
    iota_a = jax.lax.broadcasted_iota(jnp.int32, (_BLK, _A), 1)
    i1 = jnp.argmax(logits, axis=1, keepdims=True)
    l2 = jnp.where(iota_a == i1, -jnp.inf, logits)
    i2 = jnp.argmax(l2, axis=1, keepdims=True)
    selmat = jnp.logical_or(iota_a == i1, iota_a == i2).astype(jnp.float32)
    sel = [selmat[:, a : a + 1] for a in range(_A)]
    h = hall[:, : _A * _R] + b1_ref[...]
    h = 0.5 * h * (1.0 + jax.lax.erf(h * 0.7071067811865476))
    hm = jnp.concatenate(
        [h[:, a * _R : (a + 1) * _R] * sel[a] for a in range(_A)], axis=1
    ).astype(jnp.bfloat16)
    y = jnp.dot(hm, w2f_ref[...], preferred_element_type=jnp.float32)
    bias = jnp.dot(selmat, b2_ref[...], preferred_element_type=jnp.float32)
    out_ref[...] = y + bias


@jax.jit
def _fused(x2d, w1a, b12, bg2, w2f, b2):
    grid = (_N // _BLK,)
    return pl.pallas_call(
        _fused_body,
        grid=grid,
        in_specs=[
            pl.BlockSpec((_BLK, _H), lambda i: (i, 0)),
            pl.BlockSpec((_W1ROWS, _H), lambda i: (0, 0)),
            pl.BlockSpec((1, _A * _R), lambda i: (0, 0)),
            pl.BlockSpec((1, _A), lambda i: (0, 0)),
            pl.BlockSpec((_A * _R, _H), lambda i: (0, 0)),
            pl.BlockSpec((_A, _H), lambda i: (0, 0)),
        ],
        out_specs=pl.BlockSpec((_BLK, _H), lambda i: (i, 0)),
        out_shape=jax.ShapeDtypeStruct((_N, _H), jnp.float32),
        compiler_params=pltpu.CompilerParams(
            dimension_semantics=("arbitrary",),
        ),
    )(x2d, w1a, b12, bg2, w2f, b2)


def kernel(input_tensor, W1, b1, W2, b2, Wg, bg):
    x2d = input_tensor.reshape(_N, _H)
    w1a = jnp.concatenate([W1, Wg], axis=0).astype(jnp.bfloat16)
    b12 = b1.reshape(1, _A * _R)
    bg2 = bg.reshape(1, _A)
    scale = 2.0 / _K
    w2f = (W2.transpose(0, 2, 1).reshape(_A * _R, _H) * scale).astype(jnp.bfloat16)
    y = _fused(x2d, w1a, b12, bg2, w2f, b2 * scale)
    return y.reshape(_B, _L, _H)


# speedup vs baseline: 1.1445x; 1.1445x over previous
import jax
import jax.numpy as jnp
from jax.experimental import pallas as pl
from jax.experimental.pallas import tpu as pltpu

_B, _L, _H = 2, 2048, 1024
_A, _R, _K = 8, 128, 2
_N = _B * _L
_BLK = 512


def _fused_body(x_ref, wg_ref, b1_ref, bg_ref, w1_ref, w2f_ref, b2_ref, out_ref):
    xb = x_ref[...].astype(jnp.bfloat16)
    logits = jax.lax.dot_general(
        xb, wg_ref[...],
        dimension_numbers=(((1,), (1,)), ((), ())),
        preferred_element_type=jnp.float32,
    ) + bg_ref[...]
    iota_a = jax.lax.broadcasted_iota(jnp.int32, (_BLK, _A), 1)
    i1 = jnp.argmax(logits, axis=1, keepdims=True)
    l2 = jnp.where(iota_a == i1, -jnp.inf, logits)
    i2 = jnp.argmax(l2, axis=1, keepdims=True)
    selmat = jnp.logical_or(iota_a == i1, iota_a == i2).astype(jnp.float32)
    sel = [selmat[:, a : a + 1] for a in range(_A)]
    h = jax.lax.dot_general(
        xb, w1_ref[...],
        dimension_numbers=(((1,), (1,)), ((), ())),
        preferred_element_type=jnp.float32,
    ) + b1_ref[...]
    h = 0.5 * h * (1.0 + jax.lax.erf(h * 0.7071067811865476))
    hm = jnp.concatenate(
        [h[:, a * _R : (a + 1) * _R] * sel[a] for a in range(_A)], axis=1
    ).astype(jnp.bfloat16)
    y = jnp.dot(hm, w2f_ref[...], preferred_element_type=jnp.float32)
    bias = jnp.dot(selmat, b2_ref[...], preferred_element_type=jnp.float32)
    out_ref[...] = y + bias


@jax.jit
def _fused(x2d, wg, b12, bg2, w1, w2f, b2):
    grid = (_N // _BLK,)
    return pl.pallas_call(
        _fused_body,
        grid=grid,
        in_specs=[
            pl.BlockSpec((_BLK, _H), lambda i: (i, 0)),
            pl.BlockSpec((_A, _H), lambda i: (0, 0)),
            pl.BlockSpec((1, _A * _R), lambda i: (0, 0)),
            pl.BlockSpec((1, _A), lambda i: (0, 0)),
            pl.BlockSpec((_A * _R, _H), lambda i: (0, 0)),
            pl.BlockSpec((_A * _R, _H), lambda i: (0, 0)),
            pl.BlockSpec((_A, _H), lambda i: (0, 0)),
        ],
        out_specs=pl.BlockSpec((_BLK, _H), lambda i: (i, 0)),
        out_shape=jax.ShapeDtypeStruct((_N, _H), jnp.float32),
        compiler_params=pltpu.CompilerParams(
            dimension_semantics=("arbitrary",),
        ),
    )(x2d, wg, b12, bg2, w1, w2f, b2)


def kernel(input_tensor, W1, b1, W2, b2, Wg, bg):
    x2d = input_tensor.reshape(_N, _H)
    wg = Wg.astype(jnp.bfloat16)
    w1 = W1.astype(jnp.bfloat16)
    b12 = b1.reshape(1, _A * _R)
    bg2 = bg.reshape(1, _A)
    scale = 2.0 / _K
    w2f = (W2.transpose(0, 2, 1).reshape(_A * _R, _H) * scale).astype(jnp.bfloat16)
    y = _fused(x2d, wg, b12, bg2, w1, w2f, b2 * scale)
    return y.reshape(_B, _L, _H)
